# Initial kernel scaffold; baseline (speedup 1.0000x reference)
import jax
import jax.numpy as jnp
from jax.experimental import pallas as pl


def kernel(x, edge_index, edge_weight, W, b, ew1, eb1, ew2, eb2, ew3, eb3, rw1, rb1, rw2, rb2, rw3, rb3):
    raise NotImplementedError("write your pallas kernel here")


# speedup vs baseline: 8.0910x; 8.0910x over previous
import functools
import jax
import jax.numpy as jnp
from jax import lax
from jax.experimental import pallas as pl
from jax.experimental.pallas import tpu as pltpu
from jax.experimental.pallas import tpu_sc as plsc

N = 100000
E = 1600000
HEC = 4
CHUNK = 12800
NBKT = 8
NBKT_PAD = 16
NC = 2
NS = 16
NW = NC * NS
EPT = E // NW
SCAN_BLK = 2000
SCAN_NBLK = EPT // SCAN_BLK
SCAN_NVEC = SCAN_BLK // 16
BATCH = 128
CAP = E + NW * NBKT * BATCH
ABUF = 144
RPT = CHUNK // NS
RZ = 32
NPAD = NBKT * CHUNK

_mesh = lambda: plsc.VectorSubcoreMesh(core_axis_name="c", subcore_axis_name="s")


def _hist_body(dst_hbm, out_hbm, dstbuf, cnttab):
    c = lax.axis_index("c")
    s = lax.axis_index("s")
    wid = s * NC + c
    zeros16 = jnp.zeros((16,), jnp.int32)
    for i in range(16):
        cnttab[pl.ds(i * 16, 16)] = zeros16
    ones16 = jnp.ones((16,), jnp.int32)
    iota16 = lax.iota(jnp.int32, 16)
    base = wid * EPT

    def blk_loop(blk, carry):
        pltpu.sync_copy(
            dst_hbm.at[pl.ds(pl.multiple_of(base + blk * SCAN_BLK, 8), SCAN_BLK)],
            dstbuf)

        def body(v, carry2):
            dv = dstbuf[pl.ds(pl.multiple_of(v * 16, 8), 16)]
            bid = dv // CHUNK
            idx = bid * 16 + iota16
            plsc.addupdate_scatter(cnttab, [idx], ones16)
            return carry2

        return lax.fori_loop(0, SCAN_NVEC, body, carry)

    lax.fori_loop(0, SCAN_NBLK, blk_loop, 0)
    pltpu.sync_copy(cnttab, out_hbm.at[pl.ds(wid * 256, 256)])


def _hist(dst):
    f = pl.kernel(
        _hist_body,
        out_type=[jax.ShapeDtypeStruct((NW * 256,), jnp.int32)],
        mesh=_mesh(),
        compiler_params=pltpu.CompilerParams(needs_layout_passes=False),
        scratch_types=[
            pltpu.VMEM((SCAN_BLK,), jnp.int32),
            pltpu.VMEM((256,), jnp.int32),
        ],
    )
    return f(dst)[0]


def _bin_body(src_hbm, dst_hbm, ewt_hbm, segoff_hbm,
              bsrc_hbm, bdst_hbm, bew_hbm,
              srcbuf, dstbuf, e0buf, e1buf, e2buf, e3buf,
              absrc, abdst, ae0, ae1, ae2, ae3,
              segv, cnt_s, woff_s):
    c = lax.axis_index("c")
    s = lax.axis_index("s")
    wid = s * NC + c
    pltpu.sync_copy(segoff_hbm, segv)
    iota16 = lax.iota(jnp.int32, 16)
    segrow = segv[pl.ds(pl.multiple_of(wid * 16, 8), 16)]
    for b in range(NBKT):
        cnt_s[b] = 0
        woff_s[b] = segrow[b]
    base = wid * EPT
    ebufs = (e0buf, e1buf, e2buf, e3buf)
    abufs = (absrc, abdst, ae0, ae1, ae2, ae3)

    def flush(b, n):
        w0 = pl.multiple_of(woff_s[b], 8)
        pltpu.sync_copy(absrc.at[pl.ds(b * ABUF, BATCH)],
                        bsrc_hbm.at[pl.ds(w0, BATCH)])
        pltpu.sync_copy(abdst.at[pl.ds(b * ABUF, BATCH)],
                        bdst_hbm.at[pl.ds(w0, BATCH)])
        for ch in range(4):
            pltpu.sync_copy(abufs[2 + ch].at[pl.ds(b * ABUF, BATCH)],
                            bew_hbm.at[pl.ds(ch * CAP + w0, BATCH)])
        woff_s[b] = w0 + BATCH

    def blk_loop(blk, carry0):
        off = pl.multiple_of(base + blk * SCAN_BLK, 8)
        pltpu.sync_copy(src_hbm.at[pl.ds(off, SCAN_BLK)], srcbuf)
        pltpu.sync_copy(dst_hbm.at[pl.ds(off, SCAN_BLK)], dstbuf)
        for ch in range(4):
            pltpu.sync_copy(ewt_hbm.at[pl.ds(pl.multiple_of(ch * E + off, 8),
                                             SCAN_BLK)], ebufs[ch])

        def body(v, carry):
            voff = pl.multiple_of(v * 16, 8)
            dv = dstbuf[pl.ds(voff, 16)]
            sv = srcbuf[pl.ds(voff, 16)]
            evs = [eb[pl.ds(voff, 16)] for eb in ebufs]
            bid = dv // CHUNK
            drel = dv - bid * CHUNK
            vals = (sv, drel, evs[0], evs[1], evs[2], evs[3])
            for b in range(NBKT):
                m = bid == b
                mi = m.astype(jnp.int32)
                incl = plsc.cumsum(mi)
                excl = incl - mi
                c0 = cnt_s[b]
                idx = excl + (b * ABUF + c0)
                for a in range(6):
                    plsc.store_scatter(abufs[a], [idx], vals[a], mask=m)
                pc = incl[15]
                c1 = c0 + pc

                @pl.when(c1 >= BATCH)
                def _():
                    flush(b, BATCH)
                    for a in range(6):
                        tail = abufs[a][pl.ds(b * ABUF + BATCH, 16)]
                        abufs[a][pl.ds(b * ABUF, 16)] = tail

                cnt_s[b] = jnp.where(c1 >= BATCH, c1 - BATCH, c1)
            return carry

        return lax.fori_loop(0, SCAN_NVEC, body, carry0)

    lax.fori_loop(0, SCAN_NBLK, blk_loop, 0)

    zi = jnp.zeros((16,), jnp.int32)
    zf = jnp.zeros((16,), jnp.float32)
    for b in range(NBKT):
        c0 = cnt_s[b]

        @pl.when(c0 > 0)
        def _():
            for j in range(BATCH // 16):
                keep = (j * 16 + iota16) < c0
                for a in range(6):
                    cur = abufs[a][pl.ds(b * ABUF + j * 16, 16)]
                    pad = zi if a < 2 else zf
                    abufs[a][pl.ds(b * ABUF + j * 16, 16)] = jnp.where(keep, cur, pad)
            flush(b, BATCH)


def _binning(src, dst, ewt_flat, segoff):
    f = pl.kernel(
        _bin_body,
        out_type=[
            jax.ShapeDtypeStruct((CAP,), jnp.int32),
            jax.ShapeDtypeStruct((CAP,), jnp.int32),
            jax.ShapeDtypeStruct((4 * CAP,), jnp.float32),
        ],
        mesh=_mesh(),
        compiler_params=pltpu.CompilerParams(needs_layout_passes=False),
        scratch_types=[
            pltpu.VMEM((SCAN_BLK,), jnp.int32),
            pltpu.VMEM((SCAN_BLK,), jnp.int32),
            pltpu.VMEM((SCAN_BLK,), jnp.float32),
            pltpu.VMEM((SCAN_BLK,), jnp.float32),
            pltpu.VMEM((SCAN_BLK,), jnp.float32),
            pltpu.VMEM((SCAN_BLK,), jnp.float32),
            pltpu.VMEM((NBKT * ABUF,), jnp.int32),
            pltpu.VMEM((NBKT * ABUF,), jnp.int32),
            pltpu.VMEM((NBKT * ABUF,), jnp.float32),
            pltpu.VMEM((NBKT * ABUF,), jnp.float32),
            pltpu.VMEM((NBKT * ABUF,), jnp.float32),
            pltpu.VMEM((NBKT * ABUF,), jnp.float32),
            pltpu.VMEM((NBKT_PAD * NW,), jnp.int32),
            pltpu.SMEM((NBKT,), jnp.int32),
            pltpu.SMEM((NBKT,), jnp.int32),
        ],
    )
    return f(src, dst, ewt_flat, segoff)


def _agg_body(xl_hbm, bsrc_hbm, bdst_hbm, bew_hbm, meta_hbm, out_hbm,
              srcv, dstv, e0, e1, e2, e3, rows, zbuf, metav, acc, gsem):
    c = lax.axis_index("c")
    s = lax.axis_index("s")
    pltpu.sync_copy(meta_hbm, metav)
    zrow = jnp.zeros((16,), jnp.float32)

    def zb(i, carry):
        zbuf[i // 8, pl.ds((i % 8) * 16, 16)] = zrow
        return carry

    lax.fori_loop(0, RZ * 8, zb, 0)
    ebufs = (e0, e1, e2, e3)

    for p in range(NBKT // NC):
        b = p * NC + c
        for z in range(RPT // RZ):
            pltpu.sync_copy(zbuf, acc.at[pl.ds(s * RPT + z * RZ, RZ), :])
        plsc.subcore_barrier()

        bstart = metav[pl.ds(pl.multiple_of(b * 8, 8), 16)][0]
        nb = metav[pl.ds(pl.multiple_of(128 + b * 8, 8), 16)][0]
        nmine = jnp.maximum((nb - s + NS - 1) // NS, 0)

        def blk_body(i, carry):
            off = pl.multiple_of(bstart + (s + i * NS) * BATCH, 8)
            pltpu.sync_copy(bsrc_hbm.at[pl.ds(off, BATCH)], srcv)
            pltpu.sync_copy(bdst_hbm.at[pl.ds(off, BATCH)], dstv)
            for ch in range(4):
                pltpu.sync_copy(bew_hbm.at[pl.ds(ch * CAP + off, BATCH)],
                                ebufs[ch].at[pl.ds(0, BATCH)])
            pltpu.async_copy(xl_hbm.at[srcv], rows, gsem).wait()

            def scale(kk, carry2):
                koff = pl.multiple_of(kk * 16, 8)
                w0v = e0[pl.ds(koff, 16)]
                w1v = e1[pl.ds(koff, 16)]
                w2v = e2[pl.ds(koff, 16)]
                w3v = e3[pl.ds(koff, 16)]
                for j in range(16):
                    k = kk * 16 + j
                    ws = (w0v[j], w1v[j], w2v[j], w3v[j])
                    for u in range(8):
                        r = rows[k, pl.ds(u * 16, 16)]
                        rows[k, pl.ds(u * 16, 16)] = r * ws[u // 2]
                return carry2

            lax.fori_loop(0, BATCH // 16, scale, 0)
            pltpu.sync_copy(rows, acc.at[dstv], add=True)
            return carry

        lax.fori_loop(0, nmine, blk_body, 0)
        plsc.subcore_barrier()
        for z in range(RPT // RZ):
            r0 = s * RPT + z * RZ
            pltpu.sync_copy(acc.at[pl.ds(r0, RZ), :],
                            out_hbm.at[pl.ds(b * CHUNK + r0, RZ), :])
        plsc.subcore_barrier()


def _aggregate(xl, bsrc, bdst, bew_flat, meta):
    f = pl.kernel(
        _agg_body,
        out_type=[jax.ShapeDtypeStruct((NPAD, 128), jnp.float32)],
        mesh=_mesh(),
        compiler_params=pltpu.CompilerParams(needs_layout_passes=False),
        scratch_types=[
            pltpu.VMEM((BATCH,), jnp.int32),
            pltpu.VMEM((BATCH,), jnp.int32),
            pltpu.VMEM((BATCH + 16,), jnp.float32),
            pltpu.VMEM((BATCH + 16,), jnp.float32),
            pltpu.VMEM((BATCH + 16,), jnp.float32),
            pltpu.VMEM((BATCH + 16,), jnp.float32),
            pltpu.VMEM((BATCH, 128), jnp.float32),
            pltpu.VMEM((RZ, 128), jnp.float32),
            pltpu.VMEM((272,), jnp.int32),
            pltpu.VMEM_SHARED((CHUNK, 128), jnp.float32),
            pltpu.SemaphoreType.DMA,
        ],
    )
    return f(xl, bsrc, bdst, bew_flat, meta)[0]


def _edge_mlp_body(ew_ref, w1_ref, b1_ref, w2p_ref, b2p_ref, w3p_ref, b3_ref,
                   out_ref):
    h = ew_ref[...]
    w1 = w1_ref[...]
    h1 = jnp.zeros((h.shape[0], 40), jnp.float32)
    for cc in range(4):
        h1 = h1 + h[:, cc:cc + 1] * w1[cc:cc + 1, :]
    h1 = jax.nn.relu(h1 + b1_ref[...])
    h2 = jnp.dot(h1, w2p_ref[...], preferred_element_type=jnp.float32)
    h2 = jax.nn.relu(h2 + b2p_ref[...])
    t = lax.dot_general(w3p_ref[...], h2, (((0,), (1,)), ((), ())),
                        preferred_element_type=jnp.float32)
    out_ref[...] = t[0:4, :] + b3_ref[...]


def _edge_mlp(ew, ew1, eb1, ew2, eb2, ew3, eb3):
    BE = 3200
    w2p = jnp.zeros((40, 128), jnp.float32).at[:, :40].set(ew2)
    b2p = jnp.zeros((1, 128), jnp.float32).at[:, :40].set(eb2[None, :])
    w3p = jnp.zeros((128, 8), jnp.float32).at[:40, :4].set(ew3)
    grid = E // BE
    return pl.pallas_call(
        _edge_mlp_body,
        grid=(grid,),
        in_specs=[
            pl.BlockSpec((BE, 4), lambda i: (i, 0)),
            pl.BlockSpec((4, 40), lambda i: (0, 0)),
            pl.BlockSpec((1, 40), lambda i: (0, 0)),
            pl.BlockSpec((40, 128), lambda i: (0, 0)),
            pl.BlockSpec((1, 128), lambda i: (0, 0)),
            pl.BlockSpec((128, 8), lambda i: (0, 0)),
            pl.BlockSpec((4, 1), lambda i: (0, 0)),
        ],
        out_specs=pl.BlockSpec((4, BE), lambda i: (0, i)),
        out_shape=jax.ShapeDtypeStruct((4, E), jnp.float32),
    )(ew, ew1, eb1[None, :], w2p, b2p, w3p, eb3[:, None])


def _xw_body(first, x_ref, w_ref, bv_ref, o_ref):
    xb = x_ref[...]
    if not first:
        xb = jax.nn.relu(xb + bv_ref[...])
    o_ref[...] = jnp.dot(xb, w_ref[...], preferred_element_type=jnp.float32)


def _xw(x, wl, bv, first):
    BR = 2000
    return pl.pallas_call(
        functools.partial(_xw_body, first),
        grid=(N // BR,),
        in_specs=[
            pl.BlockSpec((BR, 128), lambda i: (i, 0)),
            pl.BlockSpec((128, 128), lambda i: (0, 0)),
            pl.BlockSpec((1, 128), lambda i: (0, 0)),
        ],
        out_specs=pl.BlockSpec((BR, 128), lambda i: (i, 0)),
        out_shape=jax.ShapeDtypeStruct((N, 128), jnp.float32),
    )(x, wl, bv)


def _readout_body(a_ref, bv_ref, w1_ref, b1_ref, w2_ref, b2_ref, w3r_ref,
                  b3_ref, o_ref):
    x = jax.nn.relu(a_ref[...] + bv_ref[...])
    y = jax.nn.relu(jnp.dot(x, w1_ref[...], preferred_element_type=jnp.float32)
                    + b1_ref[...])
    y = jax.nn.relu(jnp.dot(y, w2_ref[...], preferred_element_type=jnp.float32)
                    + b2_ref[...])
    y3 = jnp.sum(y * w3r_ref[...], axis=1, keepdims=True) + b3_ref[...]
    o_ref[...] = jnp.where(y3 > 0, y3, jnp.exp(y3) - 1.0) + 1.001


def _readout(agg, bv, rw1, rb1, rw2, rb2, rw3, rb3):
    BR = 2000
    return pl.pallas_call(
        _readout_body,
        grid=(N // BR,),
        in_specs=[
            pl.BlockSpec((BR, 128), lambda i: (i, 0)),
            pl.BlockSpec((1, 128), lambda i: (0, 0)),
            pl.BlockSpec((128, 128), lambda i: (0, 0)),
            pl.BlockSpec((1, 128), lambda i: (0, 0)),
            pl.BlockSpec((128, 128), lambda i: (0, 0)),
            pl.BlockSpec((1, 128), lambda i: (0, 0)),
            pl.BlockSpec((1, 128), lambda i: (0, 0)),
            pl.BlockSpec((1, 1), lambda i: (0, 0)),
        ],
        out_specs=pl.BlockSpec((BR, 1), lambda i: (i, 0)),
        out_shape=jax.ShapeDtypeStruct((N, 1), jnp.float32),
    )(agg, bv, rw1, rb1[None, :], rw2, rb2[None, :], rw3.reshape(1, 128),
      rb3.reshape(1, 1))


def kernel(x, edge_index, edge_weight, W, b, ew1, eb1, ew2, eb2, ew3, eb3,
           rw1, rb1, rw2, rb2, rw3, rb3):
    src = edge_index[0]
    dst = edge_index[1]

    ewt = _edge_mlp(edge_weight, ew1, eb1, ew2, eb2, ew3, eb3)
    ewt_flat = ewt.reshape(4 * E)

    counts_flat = _hist(dst)
    counts = counts_flat.reshape(NW, 16, 16).sum(-1)
    cnt_bt = counts.T.astype(jnp.int32)
    padcnt = ((cnt_bt + (BATCH - 1)) // BATCH) * BATCH
    flat = padcnt.reshape(-1)
    csum = jnp.cumsum(flat)
    segoff = jnp.concatenate([jnp.zeros((1,), jnp.int32),
                              csum[:-1].astype(jnp.int32)])
    bstart = segoff.reshape(NBKT_PAD, NW)[:, 0]
    totblk = (padcnt.sum(1) // BATCH).astype(jnp.int32)
    idx16 = jnp.arange(16) * 8
    meta = (jnp.zeros((272,), jnp.int32)
            .at[idx16].set(bstart)
            .at[128 + idx16].set(totblk))
    seg_tm = segoff.reshape(NBKT_PAD, NW).T.reshape(-1)

    bsrc, bdst, bew_flat = _binning(src, dst, ewt_flat, seg_tm)

    wcat = jnp.transpose(W, (0, 2, 1, 3)).reshape(3, 128, 128)
    bvec = b.reshape(3, 128)
    cur = x
    for l in range(3):
        bv = bvec[l - 1][None, :] if l > 0 else jnp.zeros((1, 128), jnp.float32)
        xl = _xw(cur, wcat[l], bv, first=(l == 0))
        cur = _aggregate(xl, bsrc, bdst, bew_flat, meta)

    return _readout(cur, bvec[2][None, :], rw1, rb1, rw2, rb2, rw3, rb3)


# speedup vs baseline: 9.2157x; 1.1390x over previous
import functools
import jax
import jax.numpy as jnp
from jax import lax
from jax.experimental import pallas as pl
from jax.experimental.pallas import tpu as pltpu
from jax.experimental.pallas import tpu_sc as plsc

N = 100000
E = 1600000
HEC = 4
CHUNK = 10240
NBKT = 10
NBKT_PAD = 16
NC = 2
NS = 16
NW = NC * NS
EPT = E // NW
SCAN_BLK = 2000
SCAN_NBLK = EPT // SCAN_BLK
SCAN_NVEC = SCAN_BLK // 16
BATCH = 128
CAP = E + NW * NBKT * BATCH
ABUF = 144
RPT = CHUNK // NS
RZ = 32
NPAD = NBKT * CHUNK

_mesh = lambda: plsc.VectorSubcoreMesh(core_axis_name="c", subcore_axis_name="s")


def _hist_body(dst_hbm, out_hbm, dstbuf, cnttab):
    c = lax.axis_index("c")
    s = lax.axis_index("s")
    wid = s * NC + c
    zeros16 = jnp.zeros((16,), jnp.int32)
    for i in range(16):
        cnttab[pl.ds(i * 16, 16)] = zeros16
    ones16 = jnp.ones((16,), jnp.int32)
    iota16 = lax.iota(jnp.int32, 16)
    base = wid * EPT

    def blk_loop(blk, carry):
        pltpu.sync_copy(
            dst_hbm.at[pl.ds(pl.multiple_of(base + blk * SCAN_BLK, 8), SCAN_BLK)],
            dstbuf)

        def body(v, carry2):
            dv = dstbuf[pl.ds(pl.multiple_of(v * 16, 8), 16)]
            bid = dv // CHUNK
            idx = bid * 16 + iota16
            plsc.addupdate_scatter(cnttab, [idx], ones16)
            return carry2

        return lax.fori_loop(0, SCAN_NVEC, body, carry)

    lax.fori_loop(0, SCAN_NBLK, blk_loop, 0)
    pltpu.sync_copy(cnttab, out_hbm.at[pl.ds(wid * 256, 256)])


def _hist(dst):
    f = pl.kernel(
        _hist_body,
        out_type=[jax.ShapeDtypeStruct((NW * 256,), jnp.int32)],
        mesh=_mesh(),
        compiler_params=pltpu.CompilerParams(needs_layout_passes=False),
        scratch_types=[
            pltpu.VMEM((SCAN_BLK,), jnp.int32),
            pltpu.VMEM((256,), jnp.int32),
        ],
    )
    return f(dst)[0]


def _bin_body(src_hbm, dst_hbm, ewt_hbm, segoff_hbm,
              bsrc_hbm, bdst_hbm, bew_hbm,
              srcbuf, dstbuf, e0buf, e1buf, e2buf, e3buf,
              absrc, abdst, ae0, ae1, ae2, ae3,
              segv, cnt_s, woff_s):
    c = lax.axis_index("c")
    s = lax.axis_index("s")
    wid = s * NC + c
    pltpu.sync_copy(segoff_hbm, segv)
    iota16 = lax.iota(jnp.int32, 16)
    segrow = segv[pl.ds(pl.multiple_of(wid * 16, 8), 16)]
    for b in range(NBKT):
        cnt_s[b] = 0
        woff_s[b] = segrow[b]
    base = wid * EPT
    ebufs = (e0buf, e1buf, e2buf, e3buf)
    abufs = (absrc, abdst, ae0, ae1, ae2, ae3)

    def flush(b, n):
        w0 = pl.multiple_of(woff_s[b], 8)
        pltpu.sync_copy(absrc.at[pl.ds(b * ABUF, BATCH)],
                        bsrc_hbm.at[pl.ds(w0, BATCH)])
        pltpu.sync_copy(abdst.at[pl.ds(b * ABUF, BATCH)],
                        bdst_hbm.at[pl.ds(w0, BATCH)])
        for ch in range(4):
            pltpu.sync_copy(abufs[2 + ch].at[pl.ds(b * ABUF, BATCH)],
                            bew_hbm.at[pl.ds(ch * CAP + w0, BATCH)])
        woff_s[b] = w0 + BATCH

    def blk_loop(blk, carry0):
        off = pl.multiple_of(base + blk * SCAN_BLK, 8)
        pltpu.sync_copy(src_hbm.at[pl.ds(off, SCAN_BLK)], srcbuf)
        pltpu.sync_copy(dst_hbm.at[pl.ds(off, SCAN_BLK)], dstbuf)
        for ch in range(4):
            pltpu.sync_copy(ewt_hbm.at[pl.ds(pl.multiple_of(ch * E + off, 8),
                                             SCAN_BLK)], ebufs[ch])

        def body(v, carry):
            voff = pl.multiple_of(v * 16, 8)
            dv = dstbuf[pl.ds(voff, 16)]
            sv = srcbuf[pl.ds(voff, 16)]
            evs = [eb[pl.ds(voff, 16)] for eb in ebufs]
            bid = dv // CHUNK
            drel = dv - bid * CHUNK
            vals = (sv, drel, evs[0], evs[1], evs[2], evs[3])
            for b in range(NBKT):
                m = bid == b
                mi = m.astype(jnp.int32)
                incl = plsc.cumsum(mi)
                excl = incl - mi
                c0 = cnt_s[b]
                idx = excl + (b * ABUF + c0)
                for a in range(6):
                    plsc.store_scatter(abufs[a], [idx], vals[a], mask=m)
                pc = incl[15]
                c1 = c0 + pc

                @pl.when(c1 >= BATCH)
                def _():
                    flush(b, BATCH)
                    for a in range(6):
                        tail = abufs[a][pl.ds(b * ABUF + BATCH, 16)]
                        abufs[a][pl.ds(b * ABUF, 16)] = tail

                cnt_s[b] = jnp.where(c1 >= BATCH, c1 - BATCH, c1)
            return carry

        return lax.fori_loop(0, SCAN_NVEC, body, carry0)

    lax.fori_loop(0, SCAN_NBLK, blk_loop, 0)

    zi = jnp.zeros((16,), jnp.int32)
    zf = jnp.zeros((16,), jnp.float32)
    for b in range(NBKT):
        c0 = cnt_s[b]

        @pl.when(c0 > 0)
        def _():
            for j in range(BATCH // 16):
                keep = (j * 16 + iota16) < c0
                for a in range(6):
                    cur = abufs[a][pl.ds(b * ABUF + j * 16, 16)]
                    pad = zi if a < 2 else zf
                    abufs[a][pl.ds(b * ABUF + j * 16, 16)] = jnp.where(keep, cur, pad)
            flush(b, BATCH)


def _binning(src, dst, ewt_flat, segoff):
    f = pl.kernel(
        _bin_body,
        out_type=[
            jax.ShapeDtypeStruct((CAP,), jnp.int32),
            jax.ShapeDtypeStruct((CAP,), jnp.int32),
            jax.ShapeDtypeStruct((4 * CAP,), jnp.float32),
        ],
        mesh=_mesh(),
        compiler_params=pltpu.CompilerParams(needs_layout_passes=False),
        scratch_types=[
            pltpu.VMEM((SCAN_BLK,), jnp.int32),
            pltpu.VMEM((SCAN_BLK,), jnp.int32),
            pltpu.VMEM((SCAN_BLK,), jnp.float32),
            pltpu.VMEM((SCAN_BLK,), jnp.float32),
            pltpu.VMEM((SCAN_BLK,), jnp.float32),
            pltpu.VMEM((SCAN_BLK,), jnp.float32),
            pltpu.VMEM((NBKT * ABUF,), jnp.int32),
            pltpu.VMEM((NBKT * ABUF,), jnp.int32),
            pltpu.VMEM((NBKT * ABUF,), jnp.float32),
            pltpu.VMEM((NBKT * ABUF,), jnp.float32),
            pltpu.VMEM((NBKT * ABUF,), jnp.float32),
            pltpu.VMEM((NBKT * ABUF,), jnp.float32),
            pltpu.VMEM((NBKT_PAD * NW,), jnp.int32),
            pltpu.SMEM((NBKT,), jnp.int32),
            pltpu.SMEM((NBKT,), jnp.int32),
        ],
    )
    return f(src, dst, ewt_flat, segoff)


def _agg_body(xl_hbm, bsrc_hbm, bdst_hbm, bew_hbm, meta_hbm, out_hbm,
              srcv, dstv, e0, e1, e2, e3, rows, zbuf, metav, acc,
              gsem0, gsem1):
    c = lax.axis_index("c")
    s = lax.axis_index("s")
    pltpu.sync_copy(meta_hbm, metav)
    zrow = jnp.zeros((16,), jnp.float32)

    def zb(i, carry):
        zbuf[i // 8, pl.ds((i % 8) * 16, 16)] = zrow
        return carry

    lax.fori_loop(0, RZ * 8, zb, 0)
    ebufs = (e0, e1, e2, e3)
    gsems = (gsem0, gsem1)
    jvecs = [jnp.full((16,), j, jnp.int32) for j in range(16)]

    def smalls(i, slot):
        off = pl.multiple_of(i * BATCH, 8)
        pltpu.sync_copy(bsrc_hbm.at[pl.ds(off, BATCH)], srcv.at[slot])
        pltpu.sync_copy(bdst_hbm.at[pl.ds(off, BATCH)], dstv.at[slot])
        for ch in range(4):
            pltpu.sync_copy(bew_hbm.at[pl.ds(pl.multiple_of(ch * CAP + off, 8),
                                             BATCH)], ebufs[ch].at[slot])

    def scale_scatter(slot):
        def sc16(kk, carry2):
            koff = pl.multiple_of(kk * 16, 8)
            wv = [eb[slot, pl.ds(koff, 16)] for eb in ebufs]
            for j in range(16):
                k = kk * 16 + j
                ws = [w.at[jvecs[j]].get(mode="promise_in_bounds")
                      for w in wv]
                for u in range(8):
                    r = rows[slot, k, pl.ds(u * 16, 16)]
                    rows[slot, k, pl.ds(u * 16, 16)] = r * ws[u // 2]
            return carry2

        lax.fori_loop(0, BATCH // 16, sc16, 0)
        pltpu.sync_copy(rows.at[slot], acc.at[dstv.at[slot]], add=True)

    def pass_body(p, carry0):
        b = p * NC + c
        for z in range(RPT // RZ):
            pltpu.sync_copy(zbuf, acc.at[pl.ds(s * RPT + z * RZ, RZ), :])
        plsc.subcore_barrier()

        bstart = metav[pl.ds(pl.multiple_of(b * 8, 8), 16)][0]
        nb = metav[pl.ds(pl.multiple_of(128 + b * 8, 8), 16)][0]
        nmine = jnp.maximum((nb - s + NS - 1) // NS, 0)
        blk0 = (bstart // BATCH) + s

        @pl.when(nmine > 0)
        def _():
            smalls(blk0, 0)

        def pair_body(q, carry):
            i0 = 2 * q
            i1 = 2 * q + 1
            has1 = i1 < nmine
            pltpu.async_copy(xl_hbm.at[srcv.at[0]], rows.at[0], gsem0)

            @pl.when(has1)
            def _():
                smalls(blk0 + i1 * NS, 1)
                pltpu.async_copy(xl_hbm.at[srcv.at[1]], rows.at[1], gsem1)

            pltpu.make_async_copy(xl_hbm.at[srcv.at[0]], rows.at[0],
                                  gsem0).wait()
            scale_scatter(0)

            @pl.when(has1)
            def _():
                pltpu.make_async_copy(xl_hbm.at[srcv.at[1]], rows.at[1],
                                      gsem1).wait()
                scale_scatter(1)

            @pl.when(i1 + 1 < nmine)
            def _():
                smalls(blk0 + (i1 + 1) * NS, 0)

            return carry

        lax.fori_loop(0, (nmine + 1) // 2, pair_body, 0)
        plsc.subcore_barrier()
        for z in range(RPT // RZ):
            r0 = s * RPT + z * RZ
            pltpu.sync_copy(acc.at[pl.ds(r0, RZ), :],
                            out_hbm.at[pl.ds(b * CHUNK + r0, RZ), :])
        plsc.subcore_barrier()
        return carry0

    lax.fori_loop(0, NBKT // NC, pass_body, 0)


def _aggregate(xl, bsrc, bdst, bew_flat, meta):
    f = pl.kernel(
        _agg_body,
        out_type=[jax.ShapeDtypeStruct((NPAD, 128), jnp.float32)],
        mesh=_mesh(),
        compiler_params=pltpu.CompilerParams(needs_layout_passes=False),
        scratch_types=[
            pltpu.VMEM((2, BATCH), jnp.int32),
            pltpu.VMEM((2, BATCH), jnp.int32),
            pltpu.VMEM((2, BATCH), jnp.float32),
            pltpu.VMEM((2, BATCH), jnp.float32),
            pltpu.VMEM((2, BATCH), jnp.float32),
            pltpu.VMEM((2, BATCH), jnp.float32),
            pltpu.VMEM((2, BATCH, 128), jnp.float32),
            pltpu.VMEM((RZ, 128), jnp.float32),
            pltpu.VMEM((272,), jnp.int32),
            pltpu.VMEM_SHARED((CHUNK, 128), jnp.float32),
            pltpu.SemaphoreType.DMA,
            pltpu.SemaphoreType.DMA,
        ],
    )
    return f(xl, bsrc, bdst, bew_flat, meta)[0]


def _edge_mlp_body(ew_ref, w1_ref, b1_ref, w2p_ref, b2p_ref, w3p_ref, b3_ref,
                   out_ref):
    h = ew_ref[...]
    w1 = w1_ref[...]
    h1 = jnp.zeros((h.shape[0], 40), jnp.float32)
    for cc in range(4):
        h1 = h1 + h[:, cc:cc + 1] * w1[cc:cc + 1, :]
    h1 = jax.nn.relu(h1 + b1_ref[...])
    h2 = jnp.dot(h1, w2p_ref[...], preferred_element_type=jnp.float32)
    h2 = jax.nn.relu(h2 + b2p_ref[...])
    t = lax.dot_general(w3p_ref[...], h2, (((0,), (1,)), ((), ())),
                        preferred_element_type=jnp.float32)
    out_ref[...] = t[0:4, :] + b3_ref[...]


def _edge_mlp(ew, ew1, eb1, ew2, eb2, ew3, eb3):
    BE = 3200
    w2p = jnp.zeros((40, 128), jnp.float32).at[:, :40].set(ew2)
    b2p = jnp.zeros((1, 128), jnp.float32).at[:, :40].set(eb2[None, :])
    w3p = jnp.zeros((128, 8), jnp.float32).at[:40, :4].set(ew3)
    grid = E // BE
    return pl.pallas_call(
        _edge_mlp_body,
        grid=(grid,),
        in_specs=[
            pl.BlockSpec((BE, 4), lambda i: (i, 0)),
            pl.BlockSpec((4, 40), lambda i: (0, 0)),
            pl.BlockSpec((1, 40), lambda i: (0, 0)),
            pl.BlockSpec((40, 128), lambda i: (0, 0)),
            pl.BlockSpec((1, 128), lambda i: (0, 0)),
            pl.BlockSpec((128, 8), lambda i: (0, 0)),
            pl.BlockSpec((4, 1), lambda i: (0, 0)),
        ],
        out_specs=pl.BlockSpec((4, BE), lambda i: (0, i)),
        out_shape=jax.ShapeDtypeStruct((4, E), jnp.float32),
    )(ew, ew1, eb1[None, :], w2p, b2p, w3p, eb3[:, None])


def _xw_body(first, x_ref, w_ref, bv_ref, o_ref):
    xb = x_ref[...]
    if not first:
        xb = jax.nn.relu(xb + bv_ref[...])
    o_ref[...] = jnp.dot(xb, w_ref[...], preferred_element_type=jnp.float32)


def _xw(x, wl, bv, first):
    BR = 2000
    return pl.pallas_call(
        functools.partial(_xw_body, first),
        grid=(N // BR,),
        in_specs=[
            pl.BlockSpec((BR, 128), lambda i: (i, 0)),
            pl.BlockSpec((128, 128), lambda i: (0, 0)),
            pl.BlockSpec((1, 128), lambda i: (0, 0)),
        ],
        out_specs=pl.BlockSpec((BR, 128), lambda i: (i, 0)),
        out_shape=jax.ShapeDtypeStruct((N, 128), jnp.float32),
    )(x, wl, bv)


def _readout_body(a_ref, bv_ref, w1_ref, b1_ref, w2_ref, b2_ref, w3r_ref,
                  b3_ref, o_ref):
    x = jax.nn.relu(a_ref[...] + bv_ref[...])
    y = jax.nn.relu(jnp.dot(x, w1_ref[...], preferred_element_type=jnp.float32)
                    + b1_ref[...])
    y = jax.nn.relu(jnp.dot(y, w2_ref[...], preferred_element_type=jnp.float32)
                    + b2_ref[...])
    y3 = jnp.sum(y * w3r_ref[...], axis=1, keepdims=True) + b3_ref[...]
    o_ref[...] = jnp.where(y3 > 0, y3, jnp.exp(y3) - 1.0) + 1.001


def _readout(agg, bv, rw1, rb1, rw2, rb2, rw3, rb3):
    BR = 2000
    return pl.pallas_call(
        _readout_body,
        grid=(N // BR,),
        in_specs=[
            pl.BlockSpec((BR, 128), lambda i: (i, 0)),
            pl.BlockSpec((1, 128), lambda i: (0, 0)),
            pl.BlockSpec((128, 128), lambda i: (0, 0)),
            pl.BlockSpec((1, 128), lambda i: (0, 0)),
            pl.BlockSpec((128, 128), lambda i: (0, 0)),
            pl.BlockSpec((1, 128), lambda i: (0, 0)),
            pl.BlockSpec((1, 128), lambda i: (0, 0)),
            pl.BlockSpec((1, 1), lambda i: (0, 0)),
        ],
        out_specs=pl.BlockSpec((BR, 1), lambda i: (i, 0)),
        out_shape=jax.ShapeDtypeStruct((N, 1), jnp.float32),
    )(agg, bv, rw1, rb1[None, :], rw2, rb2[None, :], rw3.reshape(1, 128),
      rb3.reshape(1, 1))


def kernel(x, edge_index, edge_weight, W, b, ew1, eb1, ew2, eb2, ew3, eb3,
           rw1, rb1, rw2, rb2, rw3, rb3):
    src = edge_index[0]
    dst = edge_index[1]

    ewt = _edge_mlp(edge_weight, ew1, eb1, ew2, eb2, ew3, eb3)
    ewt_flat = ewt.reshape(4 * E)

    counts_flat = _hist(dst)
    counts = counts_flat.reshape(NW, 16, 16).sum(-1)
    cnt_bt = counts.T.astype(jnp.int32)
    padcnt = ((cnt_bt + (BATCH - 1)) // BATCH) * BATCH
    flat = padcnt.reshape(-1)
    csum = jnp.cumsum(flat)
    segoff = jnp.concatenate([jnp.zeros((1,), jnp.int32),
                              csum[:-1].astype(jnp.int32)])
    bstart = segoff.reshape(NBKT_PAD, NW)[:, 0]
    totblk = (padcnt.sum(1) // BATCH).astype(jnp.int32)
    idx16 = jnp.arange(16) * 8
    meta = (jnp.zeros((272,), jnp.int32)
            .at[idx16].set(bstart)
            .at[128 + idx16].set(totblk))
    seg_tm = segoff.reshape(NBKT_PAD, NW).T.reshape(-1)

    bsrc, bdst, bew_flat = _binning(src, dst, ewt_flat, seg_tm)

    wcat = jnp.transpose(W, (0, 2, 1, 3)).reshape(3, 128, 128)
    bvec = b.reshape(3, 128)
    cur = x
    for l in range(3):
        bv = bvec[l - 1][None, :] if l > 0 else jnp.zeros((1, 128), jnp.float32)
        xl = _xw(cur, wcat[l], bv, first=(l == 0))
        cur = _aggregate(xl, bsrc, bdst, bew_flat, meta)

    return _readout(cur, bvec[2][None, :], rw1, rb1, rw2, rb2, rw3, rb3)


# speedup vs baseline: 11.9381x; 1.2954x over previous
import functools
import jax
import jax.numpy as jnp
from jax import lax
from jax.experimental import pallas as pl
from jax.experimental.pallas import tpu as pltpu
from jax.experimental.pallas import tpu_sc as plsc

N = 100000
E = 1600000
HEC = 4
CHUNK = 10240
NBKT = 10
NBKT_PAD = 16
NC = 2
NS = 16
NW = NC * NS
EPT = E // NW
SCAN_BLK = 2000
SCAN_NBLK = EPT // SCAN_BLK
SCAN_NVEC = SCAN_BLK // 16
BATCH = 128
CAP = E + NW * NBKT * BATCH
ABUF = 144
CAPB = CAP // BATCH
RPT = CHUNK // NS
RZ = 32
NPAD = NBKT * CHUNK

_mesh = lambda: plsc.VectorSubcoreMesh(core_axis_name="c", subcore_axis_name="s")


def _hist_body(dst_hbm, out_hbm, dstbuf, cnttab):
    c = lax.axis_index("c")
    s = lax.axis_index("s")
    wid = s * NC + c
    zeros16 = jnp.zeros((16,), jnp.int32)
    for i in range(16):
        cnttab[pl.ds(i * 16, 16)] = zeros16
    ones16 = jnp.ones((16,), jnp.int32)
    iota16 = lax.iota(jnp.int32, 16)
    base = wid * EPT

    def blk_loop(blk, carry):
        pltpu.sync_copy(
            dst_hbm.at[pl.ds(pl.multiple_of(base + blk * SCAN_BLK, 8), SCAN_BLK)],
            dstbuf)

        def body(v, carry2):
            dv = dstbuf[pl.ds(pl.multiple_of(v * 16, 8), 16)]
            bid = dv // CHUNK
            idx = bid * 16 + iota16
            plsc.addupdate_scatter(cnttab, [idx], ones16)
            return carry2

        return lax.fori_loop(0, SCAN_NVEC, body, carry)

    lax.fori_loop(0, SCAN_NBLK, blk_loop, 0)
    pltpu.sync_copy(cnttab, out_hbm.at[pl.ds(wid * 256, 256)])


def _hist(dst):
    f = pl.kernel(
        _hist_body,
        out_type=[jax.ShapeDtypeStruct((NW * 256,), jnp.int32)],
        mesh=_mesh(),
        compiler_params=pltpu.CompilerParams(needs_layout_passes=False),
        scratch_types=[
            pltpu.VMEM((SCAN_BLK,), jnp.int32),
            pltpu.VMEM((256,), jnp.int32),
        ],
    )
    return f(dst)[0]


def _bin_body(src_hbm, dst_hbm, ewt_hbm, segoff_hbm,
              brec_hbm,
              srcbuf, dstbuf, e0buf, e1buf, e2buf, e3buf,
              abufall,
              segv, cnt_s, woff_s):
    c = lax.axis_index("c")
    s = lax.axis_index("s")
    wid = s * NC + c
    pltpu.sync_copy(segoff_hbm, segv)
    iota16 = lax.iota(jnp.int32, 16)
    segrow = segv[pl.ds(pl.multiple_of(wid * 16, 8), 16)]
    for b in range(NBKT):
        cnt_s[b] = 0
        woff_s[b] = segrow[b]
    base = wid * EPT
    ebufs = (e0buf, e1buf, e2buf, e3buf)

    def flush(b, n):
        w0 = woff_s[b]
        pltpu.sync_copy(abufall.at[pl.ds(b * 8, 8), pl.ds(0, BATCH)],
                        brec_hbm.at[w0])
        woff_s[b] = w0 + 1

    def blk_loop(blk, carry0):
        off = pl.multiple_of(base + blk * SCAN_BLK, 8)
        pltpu.sync_copy(src_hbm.at[pl.ds(off, SCAN_BLK)], srcbuf)
        pltpu.sync_copy(dst_hbm.at[pl.ds(off, SCAN_BLK)], dstbuf)
        for ch in range(4):
            pltpu.sync_copy(ewt_hbm.at[pl.ds(pl.multiple_of(ch * E + off, 8),
                                             SCAN_BLK)], ebufs[ch])

        def body(v, carry):
            voff = pl.multiple_of(v * 16, 8)
            dv = dstbuf[pl.ds(voff, 16)]
            sv = srcbuf[pl.ds(voff, 16)]
            evs = [eb[pl.ds(voff, 16)] for eb in ebufs]
            bid = dv // CHUNK
            drel = dv - bid * CHUNK
            vals = (sv, drel, plsc.bitcast(evs[0], jnp.int32),
                    plsc.bitcast(evs[1], jnp.int32),
                    plsc.bitcast(evs[2], jnp.int32),
                    plsc.bitcast(evs[3], jnp.int32))
            for b in range(NBKT):
                m = bid == b
                mi = m.astype(jnp.int32)
                incl = plsc.cumsum(mi)
                excl = incl - mi
                c0 = cnt_s[b]
                idx = excl + c0
                for a in range(6):
                    plsc.store_scatter(abufall,
                                       [jnp.full((16,), b * 8 + a, jnp.int32),
                                        idx], vals[a], mask=m)
                pc = incl[15]
                c1 = c0 + pc

                @pl.when(c1 >= BATCH)
                def _():
                    flush(b, BATCH)
                    for a in range(6):
                        tail = abufall[b * 8 + a, pl.ds(BATCH, 16)]
                        abufall[b * 8 + a, pl.ds(0, 16)] = tail

                cnt_s[b] = jnp.where(c1 >= BATCH, c1 - BATCH, c1)
            return carry

        return lax.fori_loop(0, SCAN_NVEC, body, carry0)

    lax.fori_loop(0, SCAN_NBLK, blk_loop, 0)

    zi = jnp.zeros((16,), jnp.int32)
    for b in range(NBKT):
        c0 = cnt_s[b]

        @pl.when(c0 > 0)
        def _():
            for j in range(BATCH // 16):
                keep = (j * 16 + iota16) < c0
                for a in range(6):
                    cur = abufall[b * 8 + a, pl.ds(j * 16, 16)]
                    abufall[b * 8 + a, pl.ds(j * 16, 16)] = jnp.where(keep, cur, zi)
            flush(b, BATCH)


def _binning(src, dst, ewt_flat, segoff):
    f = pl.kernel(
        _bin_body,
        out_type=[
            jax.ShapeDtypeStruct((CAPB, 8, BATCH), jnp.int32),
        ],
        mesh=_mesh(),
        compiler_params=pltpu.CompilerParams(needs_layout_passes=False),
        scratch_types=[
            pltpu.VMEM((SCAN_BLK,), jnp.int32),
            pltpu.VMEM((SCAN_BLK,), jnp.int32),
            pltpu.VMEM((SCAN_BLK,), jnp.float32),
            pltpu.VMEM((SCAN_BLK,), jnp.float32),
            pltpu.VMEM((SCAN_BLK,), jnp.float32),
            pltpu.VMEM((SCAN_BLK,), jnp.float32),
            pltpu.VMEM((NBKT * 8, ABUF), jnp.int32),
            pltpu.VMEM((NBKT_PAD * NW,), jnp.int32),
            pltpu.SMEM((NBKT,), jnp.int32),
            pltpu.SMEM((NBKT,), jnp.int32),
        ],
    )
    return f(src, dst, ewt_flat, segoff)[0]


def _agg_body(xl_hbm, brec_hbm, meta_hbm, out_hbm,
              rbuf, rows, zbuf, metav, acc,
              gsem0, gsem1):
    c = lax.axis_index("c")
    s = lax.axis_index("s")
    pltpu.sync_copy(meta_hbm, metav)
    zrow = jnp.zeros((16,), jnp.float32)

    def zb(i, carry):
        zbuf[i // 8, pl.ds((i % 8) * 16, 16)] = zrow
        return carry

    lax.fori_loop(0, RZ * 8, zb, 0)
    jvecs = [jnp.full((16,), j, jnp.int32) for j in range(16)]

    def smalls(i, slot):
        pltpu.sync_copy(brec_hbm.at[i], rbuf.at[slot])

    def scale_scatter(slot):
        def sc16(kk, carry2):
            koff = pl.multiple_of(kk * 16, 8)
            wv = [plsc.bitcast(rbuf[slot, 2 + ch, pl.ds(koff, 16)],
                               jnp.float32) for ch in range(4)]
            for j in range(16):
                k = kk * 16 + j
                ws = [w.at[jvecs[j]].get(mode="promise_in_bounds")
                      for w in wv]
                for u in range(8):
                    r = rows[slot, k, pl.ds(u * 16, 16)]
                    rows[slot, k, pl.ds(u * 16, 16)] = r * ws[u // 2]
            return carry2

        lax.fori_loop(0, BATCH // 16, sc16, 0)
        pltpu.sync_copy(rows.at[slot], acc.at[rbuf.at[slot, 1]], add=True)

    def pass_body(p, carry0):
        b = p * NC + c
        for z in range(RPT // RZ):
            pltpu.sync_copy(zbuf, acc.at[pl.ds(s * RPT + z * RZ, RZ), :])
        plsc.subcore_barrier()

        bstart = metav[pl.ds(pl.multiple_of(b * 8, 8), 16)][0]
        nb = metav[pl.ds(pl.multiple_of(128 + b * 8, 8), 16)][0]
        nmine = jnp.maximum((nb - s + NS - 1) // NS, 0)
        blk0 = bstart + s

        @pl.when(nmine > 0)
        def _():
            smalls(blk0, 0)

        def pair_body(q, carry):
            i0 = 2 * q
            i1 = 2 * q + 1
            has1 = i1 < nmine
            pltpu.async_copy(xl_hbm.at[rbuf.at[0, 0]], rows.at[0], gsem0)

            @pl.when(has1)
            def _():
                smalls(blk0 + i1 * NS, 1)
                pltpu.async_copy(xl_hbm.at[rbuf.at[1, 0]], rows.at[1], gsem1)

            pltpu.make_async_copy(xl_hbm.at[rbuf.at[0, 0]], rows.at[0],
                                  gsem0).wait()
            scale_scatter(0)

            @pl.when(has1)
            def _():
                pltpu.make_async_copy(xl_hbm.at[rbuf.at[1, 0]], rows.at[1],
                                      gsem1).wait()
                scale_scatter(1)

            @pl.when(i1 + 1 < nmine)
            def _():
                smalls(blk0 + (i1 + 1) * NS, 0)

            return carry

        lax.fori_loop(0, (nmine + 1) // 2, pair_body, 0)
        plsc.subcore_barrier()
        for z in range(RPT // RZ):
            r0 = s * RPT + z * RZ
            pltpu.sync_copy(acc.at[pl.ds(r0, RZ), :],
                            out_hbm.at[pl.ds(b * CHUNK + r0, RZ), :])
        plsc.subcore_barrier()
        return carry0

    lax.fori_loop(0, NBKT // NC, pass_body, 0)


def _aggregate(xl, brec, meta):
    f = pl.kernel(
        _agg_body,
        out_type=[jax.ShapeDtypeStruct((NPAD, 128), jnp.float32)],
        mesh=_mesh(),
        compiler_params=pltpu.CompilerParams(needs_layout_passes=False),
        scratch_types=[
            pltpu.VMEM((2, 8, BATCH), jnp.int32),
            pltpu.VMEM((2, BATCH, 128), jnp.float32),
            pltpu.VMEM((RZ, 128), jnp.float32),
            pltpu.VMEM((272,), jnp.int32),
            pltpu.VMEM_SHARED((CHUNK, 128), jnp.float32),
            pltpu.SemaphoreType.DMA,
            pltpu.SemaphoreType.DMA,
        ],
    )
    return f(xl, brec, meta)[0]


def _edge_mlp_body(ew_ref, w1_ref, b1_ref, w2p_ref, b2p_ref, w3p_ref, b3_ref,
                   out_ref):
    h = ew_ref[...]
    w1 = w1_ref[...]
    h1 = jnp.zeros((h.shape[0], 40), jnp.float32)
    for cc in range(4):
        h1 = h1 + h[:, cc:cc + 1] * w1[cc:cc + 1, :]
    h1 = jax.nn.relu(h1 + b1_ref[...])
    h2 = jnp.dot(h1, w2p_ref[...], preferred_element_type=jnp.float32)
    h2 = jax.nn.relu(h2 + b2p_ref[...])
    t = lax.dot_general(w3p_ref[...], h2, (((0,), (1,)), ((), ())),
                        preferred_element_type=jnp.float32)
    out_ref[...] = t[0:4, :] + b3_ref[...]


def _edge_mlp(ew, ew1, eb1, ew2, eb2, ew3, eb3):
    BE = 3200
    w2p = jnp.zeros((40, 128), jnp.float32).at[:, :40].set(ew2)
    b2p = jnp.zeros((1, 128), jnp.float32).at[:, :40].set(eb2[None, :])
    w3p = jnp.zeros((128, 8), jnp.float32).at[:40, :4].set(ew3)
    grid = E // BE
    return pl.pallas_call(
        _edge_mlp_body,
        grid=(grid,),
        in_specs=[
            pl.BlockSpec((BE, 4), lambda i: (i, 0)),
            pl.BlockSpec((4, 40), lambda i: (0, 0)),
            pl.BlockSpec((1, 40), lambda i: (0, 0)),
            pl.BlockSpec((40, 128), lambda i: (0, 0)),
            pl.BlockSpec((1, 128), lambda i: (0, 0)),
            pl.BlockSpec((128, 8), lambda i: (0, 0)),
            pl.BlockSpec((4, 1), lambda i: (0, 0)),
        ],
        out_specs=pl.BlockSpec((4, BE), lambda i: (0, i)),
        out_shape=jax.ShapeDtypeStruct((4, E), jnp.float32),
    )(ew, ew1, eb1[None, :], w2p, b2p, w3p, eb3[:, None])


def _xw_body(first, x_ref, w_ref, bv_ref, o_ref):
    xb = x_ref[...]
    if not first:
        xb = jax.nn.relu(xb + bv_ref[...])
    o_ref[...] = jnp.dot(xb, w_ref[...], preferred_element_type=jnp.float32)


def _xw(x, wl, bv, first):
    BR = 2000
    return pl.pallas_call(
        functools.partial(_xw_body, first),
        grid=(N // BR,),
        in_specs=[
            pl.BlockSpec((BR, 128), lambda i: (i, 0)),
            pl.BlockSpec((128, 128), lambda i: (0, 0)),
            pl.BlockSpec((1, 128), lambda i: (0, 0)),
        ],
        out_specs=pl.BlockSpec((BR, 128), lambda i: (i, 0)),
        out_shape=jax.ShapeDtypeStruct((N, 128), jnp.float32),
    )(x, wl, bv)


def _readout_body(a_ref, bv_ref, w1_ref, b1_ref, w2_ref, b2_ref, w3r_ref,
                  b3_ref, o_ref):
    x = jax.nn.relu(a_ref[...] + bv_ref[...])
    y = jax.nn.relu(jnp.dot(x, w1_ref[...], preferred_element_type=jnp.float32)
                    + b1_ref[...])
    y = jax.nn.relu(jnp.dot(y, w2_ref[...], preferred_element_type=jnp.float32)
                    + b2_ref[...])
    y3 = jnp.sum(y * w3r_ref[...], axis=1, keepdims=True) + b3_ref[...]
    o_ref[...] = jnp.where(y3 > 0, y3, jnp.exp(y3) - 1.0) + 1.001


def _readout(agg, bv, rw1, rb1, rw2, rb2, rw3, rb3):
    BR = 2000
    return pl.pallas_call(
        _readout_body,
        grid=(N // BR,),
        in_specs=[
            pl.BlockSpec((BR, 128), lambda i: (i, 0)),
            pl.BlockSpec((1, 128), lambda i: (0, 0)),
            pl.BlockSpec((128, 128), lambda i: (0, 0)),
            pl.BlockSpec((1, 128), lambda i: (0, 0)),
            pl.BlockSpec((128, 128), lambda i: (0, 0)),
            pl.BlockSpec((1, 128), lambda i: (0, 0)),
            pl.BlockSpec((1, 128), lambda i: (0, 0)),
            pl.BlockSpec((1, 1), lambda i: (0, 0)),
        ],
        out_specs=pl.BlockSpec((BR, 1), lambda i: (i, 0)),
        out_shape=jax.ShapeDtypeStruct((N, 1), jnp.float32),
    )(agg, bv, rw1, rb1[None, :], rw2, rb2[None, :], rw3.reshape(1, 128),
      rb3.reshape(1, 1))


def kernel(x, edge_index, edge_weight, W, b, ew1, eb1, ew2, eb2, ew3, eb3,
           rw1, rb1, rw2, rb2, rw3, rb3):
    src = edge_index[0]
    dst = edge_index[1]

    ewt = _edge_mlp(edge_weight, ew1, eb1, ew2, eb2, ew3, eb3)
    ewt_flat = ewt.reshape(4 * E)

    counts_flat = _hist(dst)
    counts = counts_flat.reshape(NW, 16, 16).sum(-1)
    cnt_bt = counts.T.astype(jnp.int32)
    padcnt = ((cnt_bt + (BATCH - 1)) // BATCH) * BATCH
    flat = padcnt.reshape(-1)
    csum = jnp.cumsum(flat)
    segoff = jnp.concatenate([jnp.zeros((1,), jnp.int32),
                              csum[:-1].astype(jnp.int32)])
    segblk = segoff // BATCH
    bstart = segblk.reshape(NBKT_PAD, NW)[:, 0]
    totblk = (padcnt.sum(1) // BATCH).astype(jnp.int32)
    idx16 = jnp.arange(16) * 8
    meta = (jnp.zeros((272,), jnp.int32)
            .at[idx16].set(bstart)
            .at[128 + idx16].set(totblk))
    seg_tm = segblk.reshape(NBKT_PAD, NW).T.reshape(-1)

    brec = _binning(src, dst, ewt_flat, seg_tm)

    wcat = jnp.transpose(W, (0, 2, 1, 3)).reshape(3, 128, 128)
    bvec = b.reshape(3, 128)
    cur = x
    for l in range(3):
        bv = bvec[l - 1][None, :] if l > 0 else jnp.zeros((1, 128), jnp.float32)
        xl = _xw(cur, wcat[l], bv, first=(l == 0))
        cur = _aggregate(xl, brec, meta)

    return _readout(cur, bvec[2][None, :], rw1, rb1, rw2, rb2, rw3, rb3)


# speedup vs baseline: 12.2231x; 1.0239x over previous
import functools
import jax
import jax.numpy as jnp
from jax import lax
from jax.experimental import pallas as pl
from jax.experimental.pallas import tpu as pltpu
from jax.experimental.pallas import tpu_sc as plsc

N = 100000
E = 1600000
HEC = 4
CHUNK = 10240
NBKT = 10
NBKT_PAD = 16
NC = 2
NS = 16
NW = NC * NS
EPT = E // NW
SCAN_BLK = 2000
SCAN_NBLK = EPT // SCAN_BLK
SCAN_NVEC = SCAN_BLK // 16
BATCH = 128
CAP = E + NW * NBKT * BATCH
ABUF = 144
CAPB = CAP // BATCH
RPT = CHUNK // NS
RZ = 32
NPAD = NBKT * CHUNK

_mesh = lambda: plsc.VectorSubcoreMesh(core_axis_name="c", subcore_axis_name="s")


def _hist_body(dst_hbm, out_hbm, dstbuf, cnttab):
    c = lax.axis_index("c")
    s = lax.axis_index("s")
    wid = s * NC + c
    zeros16 = jnp.zeros((16,), jnp.int32)
    for i in range(16):
        cnttab[pl.ds(i * 16, 16)] = zeros16
    ones16 = jnp.ones((16,), jnp.int32)
    iota16 = lax.iota(jnp.int32, 16)
    base = wid * EPT

    def blk_loop(blk, carry):
        pltpu.sync_copy(
            dst_hbm.at[pl.ds(pl.multiple_of(base + blk * SCAN_BLK, 8), SCAN_BLK)],
            dstbuf)

        def body(v, carry2):
            dv = dstbuf[pl.ds(pl.multiple_of(v * 16, 8), 16)]
            bid = dv // CHUNK
            idx = bid * 16 + iota16
            plsc.addupdate_scatter(cnttab, [idx], ones16)
            return carry2

        return lax.fori_loop(0, SCAN_NVEC, body, carry)

    lax.fori_loop(0, SCAN_NBLK, blk_loop, 0)
    pltpu.sync_copy(cnttab, out_hbm.at[pl.ds(wid * 256, 256)])


def _hist(dst):
    f = pl.kernel(
        _hist_body,
        out_type=[jax.ShapeDtypeStruct((NW * 256,), jnp.int32)],
        mesh=_mesh(),
        compiler_params=pltpu.CompilerParams(needs_layout_passes=False),
        scratch_types=[
            pltpu.VMEM((SCAN_BLK,), jnp.int32),
            pltpu.VMEM((256,), jnp.int32),
        ],
    )
    return f(dst)[0]


def _bin_body(src_hbm, dst_hbm, ewt_hbm, segoff_hbm,
              brec_hbm,
              srcbuf, dstbuf, e0buf, e1buf, e2buf, e3buf,
              abufall,
              segv, cnt_s, woff_s):
    c = lax.axis_index("c")
    s = lax.axis_index("s")
    wid = s * NC + c
    pltpu.sync_copy(segoff_hbm, segv)
    iota16 = lax.iota(jnp.int32, 16)
    segrow = segv[pl.ds(pl.multiple_of(wid * 16, 8), 16)]
    for b in range(NBKT):
        cnt_s[b] = 0
        woff_s[b] = segrow[b]
    base = wid * EPT
    ebufs = (e0buf, e1buf, e2buf, e3buf)

    def flush(b, n):
        w0 = woff_s[b]
        pltpu.sync_copy(abufall.at[pl.ds(b * 8, 8), pl.ds(0, BATCH)],
                        brec_hbm.at[w0])
        woff_s[b] = w0 + 1

    def blk_loop(blk, carry0):
        off = pl.multiple_of(base + blk * SCAN_BLK, 8)
        pltpu.sync_copy(src_hbm.at[pl.ds(off, SCAN_BLK)], srcbuf)
        pltpu.sync_copy(dst_hbm.at[pl.ds(off, SCAN_BLK)], dstbuf)
        for ch in range(4):
            pltpu.sync_copy(ewt_hbm.at[pl.ds(pl.multiple_of(ch * E + off, 8),
                                             SCAN_BLK)], ebufs[ch])

        def body(v, carry):
            voff = pl.multiple_of(v * 16, 8)
            dv = dstbuf[pl.ds(voff, 16)]
            sv = srcbuf[pl.ds(voff, 16)]
            evs = [eb[pl.ds(voff, 16)] for eb in ebufs]
            bid = dv // CHUNK
            drel = dv - bid * CHUNK
            vals = (sv, drel, plsc.bitcast(evs[0], jnp.int32),
                    plsc.bitcast(evs[1], jnp.int32),
                    plsc.bitcast(evs[2], jnp.int32),
                    plsc.bitcast(evs[3], jnp.int32))
            for b in range(NBKT):
                m = bid == b
                mi = m.astype(jnp.int32)
                incl = plsc.cumsum(mi)
                excl = incl - mi
                c0 = cnt_s[b]
                idx = excl + c0
                for a in range(6):
                    plsc.store_scatter(abufall,
                                       [jnp.full((16,), b * 8 + a, jnp.int32),
                                        idx], vals[a], mask=m)
                pc = incl[15]
                c1 = c0 + pc

                @pl.when(c1 >= BATCH)
                def _():
                    flush(b, BATCH)
                    for a in range(6):
                        tail = abufall[b * 8 + a, pl.ds(BATCH, 16)]
                        abufall[b * 8 + a, pl.ds(0, 16)] = tail

                cnt_s[b] = jnp.where(c1 >= BATCH, c1 - BATCH, c1)
            return carry

        return lax.fori_loop(0, SCAN_NVEC, body, carry0)

    lax.fori_loop(0, SCAN_NBLK, blk_loop, 0)

    zi = jnp.zeros((16,), jnp.int32)
    for b in range(NBKT):
        c0 = cnt_s[b]

        @pl.when(c0 > 0)
        def _():
            for j in range(BATCH // 16):
                keep = (j * 16 + iota16) < c0
                for a in range(6):
                    cur = abufall[b * 8 + a, pl.ds(j * 16, 16)]
                    abufall[b * 8 + a, pl.ds(j * 16, 16)] = jnp.where(keep, cur, zi)
            flush(b, BATCH)


def _binning(src, dst, ewt_flat, segoff):
    f = pl.kernel(
        _bin_body,
        out_type=[
            jax.ShapeDtypeStruct((CAPB, 8, BATCH), jnp.int32),
        ],
        mesh=_mesh(),
        compiler_params=pltpu.CompilerParams(needs_layout_passes=False),
        scratch_types=[
            pltpu.VMEM((SCAN_BLK,), jnp.int32),
            pltpu.VMEM((SCAN_BLK,), jnp.int32),
            pltpu.VMEM((SCAN_BLK,), jnp.float32),
            pltpu.VMEM((SCAN_BLK,), jnp.float32),
            pltpu.VMEM((SCAN_BLK,), jnp.float32),
            pltpu.VMEM((SCAN_BLK,), jnp.float32),
            pltpu.VMEM((NBKT * 8, ABUF), jnp.int32),
            pltpu.VMEM((NBKT_PAD * NW,), jnp.int32),
            pltpu.SMEM((NBKT,), jnp.int32),
            pltpu.SMEM((NBKT,), jnp.int32),
        ],
    )
    return f(src, dst, ewt_flat, segoff)[0]


def _agg_body(xl_hbm, brec_hbm, meta_hbm, out_hbm,
              rbuf, rows, zbuf, metav, acc,
              gsem0, gsem1, ssem0, ssem1):
    c = lax.axis_index("c")
    s = lax.axis_index("s")
    pltpu.sync_copy(meta_hbm, metav)
    zrow = jnp.zeros((16,), jnp.float32)

    def zb(i, carry):
        zbuf[i // 8, pl.ds((i % 8) * 16, 16)] = zrow
        return carry

    lax.fori_loop(0, RZ * 8, zb, 0)
    jvecs = [jnp.full((16,), j, jnp.int32) for j in range(16)]

    def smalls(i, slot):
        pltpu.sync_copy(brec_hbm.at[i], rbuf.at[slot])

    ssems = (ssem0, ssem1)

    def scale_scatter(slot, ssem):
        def sc16(kk, carry2):
            koff = pl.multiple_of(kk * 16, 8)
            wv = [plsc.bitcast(rbuf[slot, 2 + ch, pl.ds(koff, 16)],
                               jnp.float32) for ch in range(4)]
            for j in range(16):
                k = kk * 16 + j
                ws = [w.at[jvecs[j]].get(mode="promise_in_bounds")
                      for w in wv]
                for u in range(8):
                    r = rows[slot, k, pl.ds(u * 16, 16)]
                    rows[slot, k, pl.ds(u * 16, 16)] = r * ws[u // 2]
            return carry2

        lax.fori_loop(0, BATCH // 16, sc16, 0)
        pltpu.async_copy(rows.at[slot], acc.at[rbuf.at[slot, 1]], ssem,
                         add=True)

    def swait(slot, ssem):
        pltpu.make_async_copy(rows.at[slot], acc.at[rbuf.at[slot, 1]],
                              ssem).wait()

    def pass_body(p, carry0):
        b = p * NC + c
        for z in range(RPT // RZ):
            pltpu.sync_copy(zbuf, acc.at[pl.ds(s * RPT + z * RZ, RZ), :])
        plsc.subcore_barrier()

        bstart = metav[pl.ds(pl.multiple_of(b * 8, 8), 16)][0]
        nb = metav[pl.ds(pl.multiple_of(128 + b * 8, 8), 16)][0]
        nmine = jnp.maximum((nb - s + NS - 1) // NS, 0)
        blk0 = bstart + s

        @pl.when(nmine > 0)
        def _():
            smalls(blk0, 0)

        def pair_body(q, carry):
            i1 = 2 * q + 1
            has1 = i1 < nmine
            pltpu.async_copy(xl_hbm.at[rbuf.at[0, 0]], rows.at[0], gsem0)

            @pl.when(has1)
            def _():
                @pl.when(q > 0)
                def _():
                    swait(1, ssem1)

                smalls(blk0 + i1 * NS, 1)
                pltpu.async_copy(xl_hbm.at[rbuf.at[1, 0]], rows.at[1], gsem1)

            pltpu.make_async_copy(xl_hbm.at[rbuf.at[0, 0]], rows.at[0],
                                  gsem0).wait()
            scale_scatter(0, ssem0)

            @pl.when(has1)
            def _():
                pltpu.make_async_copy(xl_hbm.at[rbuf.at[1, 0]], rows.at[1],
                                      gsem1).wait()
                scale_scatter(1, ssem1)

            @pl.when(i1 + 1 < nmine)
            def _():
                swait(0, ssem0)
                smalls(blk0 + (i1 + 1) * NS, 0)

            return carry

        lax.fori_loop(0, (nmine + 1) // 2, pair_body, 0)

        @pl.when(nmine > 0)
        def _():
            swait(0, ssem0)

        @pl.when((nmine > 0) & (nmine % 2 == 0))
        def _():
            swait(1, ssem1)

        plsc.subcore_barrier()
        for z in range(RPT // RZ):
            r0 = s * RPT + z * RZ
            pltpu.sync_copy(acc.at[pl.ds(r0, RZ), :],
                            out_hbm.at[pl.ds(b * CHUNK + r0, RZ), :])
        plsc.subcore_barrier()
        return carry0

    lax.fori_loop(0, NBKT // NC, pass_body, 0)


def _aggregate(xl, brec, meta):
    f = pl.kernel(
        _agg_body,
        out_type=[jax.ShapeDtypeStruct((NPAD, 128), jnp.float32)],
        mesh=_mesh(),
        compiler_params=pltpu.CompilerParams(needs_layout_passes=False),
        scratch_types=[
            pltpu.VMEM((2, 8, BATCH), jnp.int32),
            pltpu.VMEM((2, BATCH, 128), jnp.float32),
            pltpu.VMEM((RZ, 128), jnp.float32),
            pltpu.VMEM((272,), jnp.int32),
            pltpu.VMEM_SHARED((CHUNK, 128), jnp.float32),
            pltpu.SemaphoreType.DMA,
            pltpu.SemaphoreType.DMA,
            pltpu.SemaphoreType.DMA,
            pltpu.SemaphoreType.DMA,
        ],
    )
    return f(xl, brec, meta)[0]


def _edge_mlp_body(ew_ref, w1_ref, b1_ref, w2p_ref, b2p_ref, w3p_ref, b3_ref,
                   out_ref):
    h = ew_ref[...]
    w1 = w1_ref[...]
    h1 = jnp.zeros((h.shape[0], 40), jnp.float32)
    for cc in range(4):
        h1 = h1 + h[:, cc:cc + 1] * w1[cc:cc + 1, :]
    h1 = jax.nn.relu(h1 + b1_ref[...])
    h2 = jnp.dot(h1, w2p_ref[...], preferred_element_type=jnp.float32)
    h2 = jax.nn.relu(h2 + b2p_ref[...])
    t = lax.dot_general(w3p_ref[...], h2, (((0,), (1,)), ((), ())),
                        preferred_element_type=jnp.float32)
    out_ref[...] = t[0:4, :] + b3_ref[...]


def _edge_mlp(ew, ew1, eb1, ew2, eb2, ew3, eb3):
    BE = 3200
    w2p = jnp.zeros((40, 128), jnp.float32).at[:, :40].set(ew2)
    b2p = jnp.zeros((1, 128), jnp.float32).at[:, :40].set(eb2[None, :])
    w3p = jnp.zeros((128, 8), jnp.float32).at[:40, :4].set(ew3)
    grid = E // BE
    return pl.pallas_call(
        _edge_mlp_body,
        grid=(grid,),
        in_specs=[
            pl.BlockSpec((BE, 4), lambda i: (i, 0)),
            pl.BlockSpec((4, 40), lambda i: (0, 0)),
            pl.BlockSpec((1, 40), lambda i: (0, 0)),
            pl.BlockSpec((40, 128), lambda i: (0, 0)),
            pl.BlockSpec((1, 128), lambda i: (0, 0)),
            pl.BlockSpec((128, 8), lambda i: (0, 0)),
            pl.BlockSpec((4, 1), lambda i: (0, 0)),
        ],
        out_specs=pl.BlockSpec((4, BE), lambda i: (0, i)),
        out_shape=jax.ShapeDtypeStruct((4, E), jnp.float32),
    )(ew, ew1, eb1[None, :], w2p, b2p, w3p, eb3[:, None])


def _xw_body(first, x_ref, w_ref, bv_ref, o_ref):
    xb = x_ref[...]
    if not first:
        xb = jax.nn.relu(xb + bv_ref[...])
    o_ref[...] = jnp.dot(xb, w_ref[...], preferred_element_type=jnp.float32)


def _xw(x, wl, bv, first):
    BR = 2000
    return pl.pallas_call(
        functools.partial(_xw_body, first),
        grid=(N // BR,),
        in_specs=[
            pl.BlockSpec((BR, 128), lambda i: (i, 0)),
            pl.BlockSpec((128, 128), lambda i: (0, 0)),
            pl.BlockSpec((1, 128), lambda i: (0, 0)),
        ],
        out_specs=pl.BlockSpec((BR, 128), lambda i: (i, 0)),
        out_shape=jax.ShapeDtypeStruct((N, 128), jnp.float32),
    )(x, wl, bv)


def _readout_body(a_ref, bv_ref, w1_ref, b1_ref, w2_ref, b2_ref, w3r_ref,
                  b3_ref, o_ref):
    x = jax.nn.relu(a_ref[...] + bv_ref[...])
    y = jax.nn.relu(jnp.dot(x, w1_ref[...], preferred_element_type=jnp.float32)
                    + b1_ref[...])
    y = jax.nn.relu(jnp.dot(y, w2_ref[...], preferred_element_type=jnp.float32)
                    + b2_ref[...])
    y3 = jnp.sum(y * w3r_ref[...], axis=1, keepdims=True) + b3_ref[...]
    o_ref[...] = jnp.where(y3 > 0, y3, jnp.exp(y3) - 1.0) + 1.001


def _readout(agg, bv, rw1, rb1, rw2, rb2, rw3, rb3):
    BR = 2000
    return pl.pallas_call(
        _readout_body,
        grid=(N // BR,),
        in_specs=[
            pl.BlockSpec((BR, 128), lambda i: (i, 0)),
            pl.BlockSpec((1, 128), lambda i: (0, 0)),
            pl.BlockSpec((128, 128), lambda i: (0, 0)),
            pl.BlockSpec((1, 128), lambda i: (0, 0)),
            pl.BlockSpec((128, 128), lambda i: (0, 0)),
            pl.BlockSpec((1, 128), lambda i: (0, 0)),
            pl.BlockSpec((1, 128), lambda i: (0, 0)),
            pl.BlockSpec((1, 1), lambda i: (0, 0)),
        ],
        out_specs=pl.BlockSpec((BR, 1), lambda i: (i, 0)),
        out_shape=jax.ShapeDtypeStruct((N, 1), jnp.float32),
    )(agg, bv, rw1, rb1[None, :], rw2, rb2[None, :], rw3.reshape(1, 128),
      rb3.reshape(1, 1))


def kernel(x, edge_index, edge_weight, W, b, ew1, eb1, ew2, eb2, ew3, eb3,
           rw1, rb1, rw2, rb2, rw3, rb3):
    src = edge_index[0]
    dst = edge_index[1]

    ewt = _edge_mlp(edge_weight, ew1, eb1, ew2, eb2, ew3, eb3)
    ewt_flat = ewt.reshape(4 * E)

    counts_flat = _hist(dst)
    counts = counts_flat.reshape(NW, 16, 16).sum(-1)
    cnt_bt = counts.T.astype(jnp.int32)
    padcnt = ((cnt_bt + (BATCH - 1)) // BATCH) * BATCH
    flat = padcnt.reshape(-1)
    csum = jnp.cumsum(flat)
    segoff = jnp.concatenate([jnp.zeros((1,), jnp.int32),
                              csum[:-1].astype(jnp.int32)])
    segblk = segoff // BATCH
    bstart = segblk.reshape(NBKT_PAD, NW)[:, 0]
    totblk = (padcnt.sum(1) // BATCH).astype(jnp.int32)
    idx16 = jnp.arange(16) * 8
    meta = (jnp.zeros((272,), jnp.int32)
            .at[idx16].set(bstart)
            .at[128 + idx16].set(totblk))
    seg_tm = segblk.reshape(NBKT_PAD, NW).T.reshape(-1)

    brec = _binning(src, dst, ewt_flat, seg_tm)

    wcat = jnp.transpose(W, (0, 2, 1, 3)).reshape(3, 128, 128)
    bvec = b.reshape(3, 128)
    cur = x
    for l in range(3):
        bv = bvec[l - 1][None, :] if l > 0 else jnp.zeros((1, 128), jnp.float32)
        xl = _xw(cur, wcat[l], bv, first=(l == 0))
        cur = _aggregate(xl, brec, meta)

    return _readout(cur, bvec[2][None, :], rw1, rb1, rw2, rb2, rw3, rb3)


# speedup vs baseline: 12.2809x; 1.0047x over previous
import functools
import jax
import jax.numpy as jnp
from jax import lax
from jax.experimental import pallas as pl
from jax.experimental.pallas import tpu as pltpu
from jax.experimental.pallas import tpu_sc as plsc

N = 100000
E = 1600000
HEC = 4
CHUNK = 10240
NBKT = 10
NBKT_PAD = 16
NC = 2
NS = 16
NW = NC * NS
EPT = E // NW
SCAN_BLK = 2000
SCAN_NBLK = EPT // SCAN_BLK
SCAN_NVEC = SCAN_BLK // 16
BATCH = 128
CAP = E + NW * NBKT * BATCH
ABUF = 144
CAPB = CAP // BATCH
RPT = CHUNK // NS
RZ = 32
NPAD = NBKT * CHUNK

_mesh = lambda: plsc.VectorSubcoreMesh(core_axis_name="c", subcore_axis_name="s")


def _hist_body(dst_hbm, out_hbm, dstbuf, cnttab):
    c = lax.axis_index("c")
    s = lax.axis_index("s")
    wid = s * NC + c
    zeros16 = jnp.zeros((16,), jnp.int32)
    for i in range(16):
        cnttab[pl.ds(i * 16, 16)] = zeros16
    ones16 = jnp.ones((16,), jnp.int32)
    iota16 = lax.iota(jnp.int32, 16)
    base = wid * EPT

    def blk_loop(blk, carry):
        pltpu.sync_copy(
            dst_hbm.at[pl.ds(pl.multiple_of(base + blk * SCAN_BLK, 8), SCAN_BLK)],
            dstbuf)

        def body(v, carry2):
            dv = dstbuf[pl.ds(pl.multiple_of(v * 16, 8), 16)]
            bid = dv // CHUNK
            idx = bid * 16 + iota16
            plsc.addupdate_scatter(cnttab, [idx], ones16)
            return carry2

        return lax.fori_loop(0, SCAN_NVEC, body, carry)

    lax.fori_loop(0, SCAN_NBLK, blk_loop, 0)
    pltpu.sync_copy(cnttab, out_hbm.at[pl.ds(wid * 256, 256)])


def _hist(dst):
    f = pl.kernel(
        _hist_body,
        out_type=[jax.ShapeDtypeStruct((NW * 256,), jnp.int32)],
        mesh=_mesh(),
        compiler_params=pltpu.CompilerParams(needs_layout_passes=False),
        scratch_types=[
            pltpu.VMEM((SCAN_BLK,), jnp.int32),
            pltpu.VMEM((256,), jnp.int32),
        ],
    )
    return f(dst)[0]


def _bin_body(src_hbm, dst_hbm, ewt_hbm, segoff_hbm,
              brec_hbm,
              srcbuf, dstbuf, e0buf, e1buf, e2buf, e3buf,
              abufall,
              segv, cnt_s, woff_s):
    c = lax.axis_index("c")
    s = lax.axis_index("s")
    wid = s * NC + c
    pltpu.sync_copy(segoff_hbm, segv)
    iota16 = lax.iota(jnp.int32, 16)
    segrow = segv[pl.ds(pl.multiple_of(wid * 16, 8), 16)]
    for b in range(NBKT):
        cnt_s[b] = 0
        woff_s[b] = segrow[b]
    base = wid * EPT
    ebufs = (e0buf, e1buf, e2buf, e3buf)

    def flush(b, n):
        w0 = woff_s[b]
        pltpu.sync_copy(abufall.at[pl.ds(b * 8, 8), pl.ds(0, BATCH)],
                        brec_hbm.at[w0])
        woff_s[b] = w0 + 1

    def blk_loop(blk, carry0):
        off = pl.multiple_of(base + blk * SCAN_BLK, 8)
        pltpu.sync_copy(src_hbm.at[pl.ds(off, SCAN_BLK)], srcbuf)
        pltpu.sync_copy(dst_hbm.at[pl.ds(off, SCAN_BLK)], dstbuf)
        for ch in range(4):
            pltpu.sync_copy(ewt_hbm.at[pl.ds(pl.multiple_of(ch * E + off, 8),
                                             SCAN_BLK)], ebufs[ch])

        def body(v, carry):
            voff = pl.multiple_of(v * 16, 8)
            dv = dstbuf[pl.ds(voff, 16)]
            sv = srcbuf[pl.ds(voff, 16)]
            evs = [eb[pl.ds(voff, 16)] for eb in ebufs]
            bid = dv // CHUNK
            drel = dv - bid * CHUNK
            vals = (sv, drel, plsc.bitcast(evs[0], jnp.int32),
                    plsc.bitcast(evs[1], jnp.int32),
                    plsc.bitcast(evs[2], jnp.int32),
                    plsc.bitcast(evs[3], jnp.int32))
            for b in range(NBKT):
                m = bid == b
                mi = m.astype(jnp.int32)
                incl = plsc.cumsum(mi)
                excl = incl - mi
                c0 = cnt_s[b]
                idx = excl + c0
                for a in range(6):
                    plsc.store_scatter(abufall,
                                       [jnp.full((16,), b * 8 + a, jnp.int32),
                                        idx], vals[a], mask=m)
                pc = incl[15]
                c1 = c0 + pc

                @pl.when(c1 >= BATCH)
                def _():
                    flush(b, BATCH)
                    for a in range(6):
                        tail = abufall[b * 8 + a, pl.ds(BATCH, 16)]
                        abufall[b * 8 + a, pl.ds(0, 16)] = tail

                cnt_s[b] = jnp.where(c1 >= BATCH, c1 - BATCH, c1)
            return carry

        return lax.fori_loop(0, SCAN_NVEC, body, carry0)

    lax.fori_loop(0, SCAN_NBLK, blk_loop, 0)

    zi = jnp.zeros((16,), jnp.int32)
    for b in range(NBKT):
        c0 = cnt_s[b]

        @pl.when(c0 > 0)
        def _():
            for j in range(BATCH // 16):
                keep = (j * 16 + iota16) < c0
                for a in range(6):
                    cur = abufall[b * 8 + a, pl.ds(j * 16, 16)]
                    abufall[b * 8 + a, pl.ds(j * 16, 16)] = jnp.where(keep, cur, zi)
            flush(b, BATCH)


def _binning(src, dst, ewt_flat, segoff):
    f = pl.kernel(
        _bin_body,
        out_type=[
            jax.ShapeDtypeStruct((CAPB, 8, BATCH), jnp.int32),
        ],
        mesh=_mesh(),
        compiler_params=pltpu.CompilerParams(needs_layout_passes=False),
        scratch_types=[
            pltpu.VMEM((SCAN_BLK,), jnp.int32),
            pltpu.VMEM((SCAN_BLK,), jnp.int32),
            pltpu.VMEM((SCAN_BLK,), jnp.float32),
            pltpu.VMEM((SCAN_BLK,), jnp.float32),
            pltpu.VMEM((SCAN_BLK,), jnp.float32),
            pltpu.VMEM((SCAN_BLK,), jnp.float32),
            pltpu.VMEM((NBKT * 8, ABUF), jnp.int32),
            pltpu.VMEM((NBKT_PAD * NW,), jnp.int32),
            pltpu.SMEM((NBKT,), jnp.int32),
            pltpu.SMEM((NBKT,), jnp.int32),
        ],
    )
    return f(src, dst, ewt_flat, segoff)[0]


def _agg_body(xl_hbm, brec_hbm, meta_hbm, out_hbm,
              rbuf, rows, zbuf, metav, acc,
              gsem0, gsem1, ssem0, ssem1):
    c = lax.axis_index("c")
    s = lax.axis_index("s")
    pltpu.sync_copy(meta_hbm, metav)
    zrow = jnp.zeros((16,), jnp.float32)

    def zb(i, carry):
        zbuf[i // 8, pl.ds((i % 8) * 16, 16)] = zrow
        return carry

    lax.fori_loop(0, RZ * 8, zb, 0)
    jvecs = [jnp.full((16,), j, jnp.int32) for j in range(16)]

    def smalls(i, slot):
        pltpu.sync_copy(brec_hbm.at[i], rbuf.at[slot])

    ssems = (ssem0, ssem1)

    def scale_scatter(slot, ssem):
        def sc16(kk, carry2):
            koff = pl.multiple_of(kk * 16, 8)
            wv = [plsc.bitcast(rbuf[slot, 2 + ch, pl.ds(koff, 16)],
                               jnp.float32) for ch in range(4)]
            for j in range(16):
                k = kk * 16 + j
                ws = [w.at[jvecs[j]].get(mode="promise_in_bounds")
                      for w in wv]
                for u in range(8):
                    r = rows[slot, k, pl.ds(u * 16, 16)]
                    rows[slot, k, pl.ds(u * 16, 16)] = r * ws[u // 2]
            return carry2

        lax.fori_loop(0, BATCH // 16, sc16, 0)
        pltpu.async_copy(rows.at[slot], acc.at[rbuf.at[slot, 1]], ssem,
                         add=True)

    def swait(slot, ssem):
        pltpu.make_async_copy(rows.at[slot], acc.at[rbuf.at[slot, 1]],
                              ssem).wait()

    def pass_body(p, carry0):
        b = p * NC + c
        for z in range(RPT // RZ):
            pltpu.sync_copy(zbuf, acc.at[pl.ds(s * RPT + z * RZ, RZ), :])
        plsc.subcore_barrier()

        bstart = metav[pl.ds(pl.multiple_of(b * 8, 8), 16)][0]
        nb = metav[pl.ds(pl.multiple_of(128 + b * 8, 8), 16)][0]
        nmine = jnp.maximum((nb - s + NS - 1) // NS, 0)
        blk0 = bstart + s

        @pl.when(nmine > 0)
        def _():
            smalls(blk0, 0)

        def pair_body(q, carry):
            i1 = 2 * q + 1
            has1 = i1 < nmine
            pltpu.async_copy(xl_hbm.at[rbuf.at[0, 0]], rows.at[0], gsem0)

            @pl.when(has1)
            def _():
                @pl.when(q > 0)
                def _():
                    swait(1, ssem1)

                smalls(blk0 + i1 * NS, 1)
                pltpu.async_copy(xl_hbm.at[rbuf.at[1, 0]], rows.at[1], gsem1)

            pltpu.make_async_copy(xl_hbm.at[rbuf.at[0, 0]], rows.at[0],
                                  gsem0).wait()
            scale_scatter(0, ssem0)

            @pl.when(has1)
            def _():
                pltpu.make_async_copy(xl_hbm.at[rbuf.at[1, 0]], rows.at[1],
                                      gsem1).wait()
                scale_scatter(1, ssem1)

            @pl.when(i1 + 1 < nmine)
            def _():
                swait(0, ssem0)
                smalls(blk0 + (i1 + 1) * NS, 0)

            return carry

        lax.fori_loop(0, (nmine + 1) // 2, pair_body, 0)

        @pl.when(nmine > 0)
        def _():
            swait(0, ssem0)

        @pl.when((nmine > 0) & (nmine % 2 == 0))
        def _():
            swait(1, ssem1)

        plsc.subcore_barrier()
        for z in range(RPT // RZ):
            r0 = s * RPT + z * RZ
            pltpu.sync_copy(acc.at[pl.ds(r0, RZ), :],
                            out_hbm.at[pl.ds(b * CHUNK + r0, RZ), :])
        plsc.subcore_barrier()
        return carry0

    lax.fori_loop(0, NBKT // NC, pass_body, 0)


def _aggregate(xl, brec, meta):
    f = pl.kernel(
        _agg_body,
        out_type=[jax.ShapeDtypeStruct((NPAD, 128), jnp.float32)],
        mesh=_mesh(),
        compiler_params=pltpu.CompilerParams(needs_layout_passes=False),
        scratch_types=[
            pltpu.VMEM((2, 8, BATCH), jnp.int32),
            pltpu.VMEM((2, BATCH, 128), jnp.float32),
            pltpu.VMEM((RZ, 128), jnp.float32),
            pltpu.VMEM((272,), jnp.int32),
            pltpu.VMEM_SHARED((CHUNK, 128), jnp.float32),
            pltpu.SemaphoreType.DMA,
            pltpu.SemaphoreType.DMA,
            pltpu.SemaphoreType.DMA,
            pltpu.SemaphoreType.DMA,
        ],
    )
    return f(xl, brec, meta)[0]


def _edge_mlp_body(ew_ref, w1_ref, b1_ref, w2p_ref, b2p_ref, w3p_ref, b3_ref,
                   out_ref):
    h = ew_ref[...]
    w1 = w1_ref[...]
    h1 = jnp.zeros((h.shape[0], 40), jnp.float32)
    for cc in range(4):
        h1 = h1 + h[:, cc:cc + 1] * w1[cc:cc + 1, :]
    h1 = jax.nn.relu(h1 + b1_ref[...])
    h2 = jnp.dot(h1, w2p_ref[...], preferred_element_type=jnp.float32)
    h2 = jax.nn.relu(h2 + b2p_ref[...])
    t = lax.dot_general(w3p_ref[...], h2, (((0,), (1,)), ((), ())),
                        preferred_element_type=jnp.float32)
    out_ref[...] = t[0:4, :] + b3_ref[...]


def _edge_mlp(ew, ew1, eb1, ew2, eb2, ew3, eb3):
    BE = 12800
    w2p = jnp.zeros((40, 128), jnp.float32).at[:, :40].set(ew2)
    b2p = jnp.zeros((1, 128), jnp.float32).at[:, :40].set(eb2[None, :])
    w3p = jnp.zeros((128, 8), jnp.float32).at[:40, :4].set(ew3)
    grid = E // BE
    return pl.pallas_call(
        _edge_mlp_body,
        grid=(grid,),
        in_specs=[
            pl.BlockSpec((BE, 4), lambda i: (i, 0)),
            pl.BlockSpec((4, 40), lambda i: (0, 0)),
            pl.BlockSpec((1, 40), lambda i: (0, 0)),
            pl.BlockSpec((40, 128), lambda i: (0, 0)),
            pl.BlockSpec((1, 128), lambda i: (0, 0)),
            pl.BlockSpec((128, 8), lambda i: (0, 0)),
            pl.BlockSpec((4, 1), lambda i: (0, 0)),
        ],
        out_specs=pl.BlockSpec((4, BE), lambda i: (0, i)),
        out_shape=jax.ShapeDtypeStruct((4, E), jnp.float32),
    )(ew, ew1, eb1[None, :], w2p, b2p, w3p, eb3[:, None])


def _xw_body(first, x_ref, w_ref, bv_ref, o_ref):
    xb = x_ref[...]
    if not first:
        xb = jax.nn.relu(xb + bv_ref[...])
    o_ref[...] = jnp.dot(xb, w_ref[...], preferred_element_type=jnp.float32)


def _xw(x, wl, bv, first):
    BR = 2000
    return pl.pallas_call(
        functools.partial(_xw_body, first),
        grid=(N // BR,),
        in_specs=[
            pl.BlockSpec((BR, 128), lambda i: (i, 0)),
            pl.BlockSpec((128, 128), lambda i: (0, 0)),
            pl.BlockSpec((1, 128), lambda i: (0, 0)),
        ],
        out_specs=pl.BlockSpec((BR, 128), lambda i: (i, 0)),
        out_shape=jax.ShapeDtypeStruct((N, 128), jnp.float32),
    )(x, wl, bv)


def _readout_body(a_ref, bv_ref, w1_ref, b1_ref, w2_ref, b2_ref, w3r_ref,
                  b3_ref, o_ref):
    x = jax.nn.relu(a_ref[...] + bv_ref[...])
    y = jax.nn.relu(jnp.dot(x, w1_ref[...], preferred_element_type=jnp.float32)
                    + b1_ref[...])
    y = jax.nn.relu(jnp.dot(y, w2_ref[...], preferred_element_type=jnp.float32)
                    + b2_ref[...])
    y3 = jnp.sum(y * w3r_ref[...], axis=1, keepdims=True) + b3_ref[...]
    o_ref[...] = jnp.where(y3 > 0, y3, jnp.exp(y3) - 1.0) + 1.001


def _readout(agg, bv, rw1, rb1, rw2, rb2, rw3, rb3):
    BR = 2000
    return pl.pallas_call(
        _readout_body,
        grid=(N // BR,),
        in_specs=[
            pl.BlockSpec((BR, 128), lambda i: (i, 0)),
            pl.BlockSpec((1, 128), lambda i: (0, 0)),
            pl.BlockSpec((128, 128), lambda i: (0, 0)),
            pl.BlockSpec((1, 128), lambda i: (0, 0)),
            pl.BlockSpec((128, 128), lambda i: (0, 0)),
            pl.BlockSpec((1, 128), lambda i: (0, 0)),
            pl.BlockSpec((1, 128), lambda i: (0, 0)),
            pl.BlockSpec((1, 1), lambda i: (0, 0)),
        ],
        out_specs=pl.BlockSpec((BR, 1), lambda i: (i, 0)),
        out_shape=jax.ShapeDtypeStruct((N, 1), jnp.float32),
    )(agg, bv, rw1, rb1[None, :], rw2, rb2[None, :], rw3.reshape(1, 128),
      rb3.reshape(1, 1))


def kernel(x, edge_index, edge_weight, W, b, ew1, eb1, ew2, eb2, ew3, eb3,
           rw1, rb1, rw2, rb2, rw3, rb3):
    src = edge_index[0]
    dst = edge_index[1]

    ewt = _edge_mlp(edge_weight, ew1, eb1, ew2, eb2, ew3, eb3)
    ewt_flat = ewt.reshape(4 * E)

    counts_flat = _hist(dst)
    counts = counts_flat.reshape(NW, 16, 16).sum(-1)
    cnt_bt = counts.T.astype(jnp.int32)
    padcnt = ((cnt_bt + (BATCH - 1)) // BATCH) * BATCH
    flat = padcnt.reshape(-1)
    csum = jnp.cumsum(flat)
    segoff = jnp.concatenate([jnp.zeros((1,), jnp.int32),
                              csum[:-1].astype(jnp.int32)])
    segblk = segoff // BATCH
    bstart = segblk.reshape(NBKT_PAD, NW)[:, 0]
    totblk = (padcnt.sum(1) // BATCH).astype(jnp.int32)
    idx16 = jnp.arange(16) * 8
    meta = (jnp.zeros((272,), jnp.int32)
            .at[idx16].set(bstart)
            .at[128 + idx16].set(totblk))
    seg_tm = segblk.reshape(NBKT_PAD, NW).T.reshape(-1)

    brec = _binning(src, dst, ewt_flat, seg_tm)

    wcat = jnp.transpose(W, (0, 2, 1, 3)).reshape(3, 128, 128)
    bvec = b.reshape(3, 128)
    cur = x
    for l in range(3):
        bv = bvec[l - 1][None, :] if l > 0 else jnp.zeros((1, 128), jnp.float32)
        xl = _xw(cur, wcat[l], bv, first=(l == 0))
        cur = _aggregate(xl, brec, meta)

    return _readout(cur, bvec[2][None, :], rw1, rb1, rw2, rb2, rw3, rb3)
